# baseline (speedup 1.0000x reference)
import jax
import jax.numpy as jnp
from jax import lax
from jax.experimental import pallas as pl
from jax.experimental.pallas import tpu as pltpu
from jax.experimental.pallas import tpu_sc as plsc

BATCH = 4096
WIDTH = 26
DIM = 64
PAIRS = 500000
NBUF = 4

_info = plsc.get_sparse_core_info()
NC, NS = _info.num_cores, _info.num_subcores
NW = NC * NS
B_PER_W = BATCH // NW
assert B_PER_W * NW == BATCH

_GOFF = (0, WIDTH - 16)


def _body(idx_hbm, table_hbm, out_hbm, idxv, pidxs, pairs, obufs, gsems, wsems):
    wid = lax.axis_index("s") * NC + lax.axis_index("c")
    bbase = wid * B_PER_W
    iota16 = jnp.arange(16, dtype=jnp.int32)

    pltpu.sync_copy(idx_hbm.at[pl.ds(bbase, B_PER_W)], idxv)

    def prep_and_gather(k, b):
        for off in _GOFF:
            iv = idxv[k, pl.ds(off, 16)]
            pidxs[b][pl.ds(off, 16)] = iv >> 1
        pltpu.async_copy(table_hbm.at[pidxs[b]], pairs[b], gsems[b])

    def process(k, b):
        pltpu.make_async_copy(table_hbm.at[pidxs[b]], pairs[b], gsems[b]).wait()
        for off in _GOFF:
            iv = idxv[k, pl.ds(off, 16)]
            r16 = off + iota16
            h16 = (iv & 1) * DIM
            for c in range(DIM):
                v = plsc.load_gather(pairs[b], [r16, h16 + c])
                plsc.store_scatter(obufs[b], [r16, jnp.full((16,), c, jnp.int32)], v)
            m = iv == 0
            nz = jnp.max(plsc.all_reduce_population_count(m))

            @pl.when(nz > 0)
            def _():
                zeros = jnp.zeros((16,), jnp.float32)

                def dcol(d, carry):
                    cid = jnp.full((16,), d, jnp.int32)
                    plsc.store_scatter(obufs[b], [r16, cid], zeros, mask=m)
                    return carry

                lax.fori_loop(0, DIM, dcol, 0)

        dst = out_hbm.at[bbase + k]
        pltpu.async_copy(obufs[b], dst, wsems[b])
        pltpu.make_async_copy(obufs[b], dst, wsems[b]).wait()

    for b in range(NBUF):
        prep_and_gather(b, b)

    def outer(i, carry):
        for b in range(NBUF):
            k = i * NBUF + b
            process(k, b)
            prep_and_gather(k + NBUF, b)
        return carry

    lax.fori_loop(0, B_PER_W // NBUF - 1, outer, 0)

    for b in range(NBUF):
        process(B_PER_W - NBUF + b, b)


def _lookup(idx, table2):
    mesh = plsc.VectorSubcoreMesh(core_axis_name="c", subcore_axis_name="s")
    scratch = [
        pltpu.VMEM((B_PER_W, WIDTH), jnp.int32),
        [pltpu.VMEM((WIDTH,), jnp.int32) for _ in range(NBUF)],
        [pltpu.VMEM((WIDTH, 2 * DIM), jnp.float32) for _ in range(NBUF)],
        [pltpu.VMEM((WIDTH, DIM), jnp.float32) for _ in range(NBUF)],
        [pltpu.SemaphoreType.DMA for _ in range(NBUF)],
        [pltpu.SemaphoreType.DMA for _ in range(NBUF)],
    ]
    k = pl.kernel(
        _body,
        mesh=mesh,
        out_type=jax.ShapeDtypeStruct((BATCH, WIDTH, DIM), jnp.float32),
        scratch_types=scratch,
        compiler_params=pltpu.CompilerParams(needs_layout_passes=False),
    )
    return k(idx, table2)


@jax.jit
def _run(idx, table):
    return _lookup(idx, table.reshape(PAIRS, 2 * DIM))


def kernel(input, table):
    return _run(input.astype(jnp.int32), table)


# speedup vs baseline: 1.3839x; 1.3839x over previous
import jax
import jax.numpy as jnp
from jax import lax
from jax.experimental import pallas as pl
from jax.experimental.pallas import tpu as pltpu
from jax.experimental.pallas import tpu_sc as plsc

BATCH = 4096
WIDTH = 26
DIM = 64
TOTAL = BATCH * WIDTH
CHUNK = 128
NBUF = 4

_info = plsc.get_sparse_core_info()
NC, NS = _info.num_cores, _info.num_subcores
NW = NC * NS
PER_W = TOTAL // NW
NSTEP = PER_W // CHUNK
assert PER_W * NW == TOTAL and NSTEP * CHUNK == PER_W


def _body(idx_hbm, table_hbm, out_hbm, idxs, rows, gsems, wsems):
    wid = lax.axis_index("s") * NC + lax.axis_index("c")
    base = wid * PER_W

    def zero_fixup(b):
        def group(g, carry):
            iv = idxs[b][pl.ds(g * 16, 16)]
            m = iv == 0
            nz = jnp.max(plsc.all_reduce_population_count(m))

            @pl.when(nz > 0)
            def _():
                rid = g * 16 + jnp.arange(16, dtype=jnp.int32)
                zeros = jnp.zeros((16,), jnp.float32)

                def dcol(d, c):
                    cid = jnp.full((16,), d, jnp.int32)
                    plsc.store_scatter(rows[b], [rid, cid], zeros, mask=m)
                    return c

                lax.fori_loop(0, DIM, dcol, 0)

            return carry

        lax.fori_loop(0, CHUNK // 16, group, 0)

    def gather(s):
        b = s % NBUF
        pltpu.sync_copy(idx_hbm.at[pl.ds(base + s * CHUNK, CHUNK)], idxs[b])
        pltpu.async_copy(table_hbm.at[idxs[b]], rows[b], gsems[b])

    for s in range(NBUF):
        gather(s)

    for s in range(NSTEP):
        b = s % NBUF
        pltpu.make_async_copy(table_hbm.at[idxs[b]], rows[b], gsems[b]).wait()
        zero_fixup(b)
        dst = out_hbm.at[pl.ds(base + s * CHUNK, CHUNK)]
        pltpu.async_copy(rows[b], dst, wsems[b])
        pltpu.make_async_copy(rows[b], dst, wsems[b]).wait()
        if s + NBUF < NSTEP:
            gather(s + NBUF)


@jax.jit
def _lookup(idx_flat, table):
    mesh = plsc.VectorSubcoreMesh(core_axis_name="c", subcore_axis_name="s")
    scratch = [
        [pltpu.VMEM((CHUNK,), jnp.int32) for _ in range(NBUF)],
        [pltpu.VMEM((CHUNK, DIM), jnp.float32) for _ in range(NBUF)],
        [pltpu.SemaphoreType.DMA for _ in range(NBUF)],
        [pltpu.SemaphoreType.DMA for _ in range(NBUF)],
    ]
    k = pl.kernel(
        _body,
        mesh=mesh,
        out_type=jax.ShapeDtypeStruct((TOTAL, DIM), jnp.float32),
        scratch_types=scratch,
        compiler_params=pltpu.CompilerParams(
            use_tc_tiling_on_sc=False, needs_layout_passes=False
        ),
    )
    return k(idx_flat, table)


def kernel(input, table):
    idx_flat = input.astype(jnp.int32).reshape(TOTAL)
    out = _lookup(idx_flat, table)
    return out.reshape(BATCH, WIDTH, DIM)
